# Initial kernel scaffold; baseline (speedup 1.0000x reference)
import jax
import jax.numpy as jnp
from jax.experimental import pallas as pl


def kernel(confidence, predicted_locations, gts, counts, anchors):
    raise NotImplementedError("write your pallas kernel here")


# speedup vs baseline: 5.9705x; 5.9705x over previous
import functools

import jax
import jax.numpy as jnp
from jax.experimental import pallas as pl
from jax.experimental.pallas import tpu as pltpu

_POS_THRESH = 0.5
_NEG_POS_RATIO = 3
_VAR_C = 0.1
_VAR_S = 0.2

_A = 8732
_G = 50
_C = 21
_R = 8
_L = 1152
_A_PAD = _R * _L


def _mbl_body(conf_ref, pred_ref, gts_ref, counts_ref, anch_ref, out_ref):
    b = pl.program_id(0)
    nb = pl.num_programs(0)
    count = counts_ref[0, 0, 0]

    ax1 = anch_ref[0]
    ay1 = anch_ref[1]
    ax2 = anch_ref[2]
    ay2 = anch_ref[3]
    acx = anch_ref[4]
    acy = anch_ref[5]
    aw = anch_ref[6]
    ah = anch_ref[7]
    aarea = anch_ref[8]

    ridx = jax.lax.broadcasted_iota(jnp.int32, (_R, _L), 0)
    cidx = jax.lax.broadcasted_iota(jnp.int32, (_R, _L), 1)
    aidx = ridx * _L + cidx
    valid_anchor = aidx < _A

    def gt_body(g, carry):
        best_iou, best_idx = carry
        gx1 = gts_ref[0, g, 0]
        gy1 = gts_ref[0, g, 1]
        gx2 = gts_ref[0, g, 2]
        gy2 = gts_ref[0, g, 3]
        valid = g < count

        lt_x = jnp.maximum(gx1, ax1)
        lt_y = jnp.maximum(gy1, ay1)
        rb_x = jnp.minimum(gx2, ax2)
        rb_y = jnp.minimum(gy2, ay2)
        inter = jnp.maximum(rb_x - lt_x, 0.0) * jnp.maximum(rb_y - lt_y, 0.0)
        garea = jnp.maximum(gx2 - gx1, 0.0) * jnp.maximum(gy2 - gy1, 0.0)
        union = garea + aarea - inter
        iou = inter / jnp.maximum(union, 1e-8)

        upd = jnp.logical_and(valid, iou > best_iou)
        best_idx = jnp.where(upd, g, best_idx)
        best_iou = jnp.where(upd, iou, best_iou)

        m = jnp.max(iou)
        astar = jnp.min(jnp.where(iou == m, aidx, jnp.int32(2**30)))
        fm = jnp.logical_and(valid, aidx == astar)
        best_idx = jnp.where(fm, g, best_idx)
        best_iou = jnp.where(fm, jnp.float32(2.0), best_iou)
        return best_iou, best_idx

    best_iou0 = jnp.full((_R, _L), -1.0, dtype=jnp.float32)
    best_idx0 = jnp.zeros((_R, _L), dtype=jnp.int32)
    best_iou, best_idx = jax.lax.fori_loop(0, _G, gt_body, (best_iou0, best_idx0))

    def gather_body(g, carry):
        lab, bx1, by1, bx2, by2 = carry
        m = best_idx == g
        lab = jnp.where(m, gts_ref[0, g, 4], lab)
        bx1 = jnp.where(m, gts_ref[0, g, 0], bx1)
        by1 = jnp.where(m, gts_ref[0, g, 1], by1)
        bx2 = jnp.where(m, gts_ref[0, g, 2], bx2)
        by2 = jnp.where(m, gts_ref[0, g, 3], by2)
        return lab, bx1, by1, bx2, by2

    z = jnp.zeros((_R, _L), dtype=jnp.float32)
    lab, bx1, by1, bx2, by2 = jax.lax.fori_loop(
        0, _G, gather_body, (z, z, z, z, z))

    label = jnp.where(best_iou < _POS_THRESH, 0.0, lab)
    pos = label > 0.0
    num_pos = jnp.sum(jnp.where(pos, 1.0, 0.0))

    gcx = (bx1 + bx2) * 0.5
    gcy = (by1 + by2) * 0.5
    gw = bx2 - bx1
    gh = by2 - by1
    t0 = (gcx - acx) / (aw * _VAR_C)
    t1 = (gcy - acy) / (ah * _VAR_C)
    t2 = jnp.log(jnp.maximum(gw, 1e-8) / aw) / _VAR_S
    t3 = jnp.log(jnp.maximum(gh, 1e-8) / ah) / _VAR_S

    loc_sum = jnp.float32(0.0)
    for j, t in enumerate((t0, t1, t2, t3)):
        d = pred_ref[0, j] - t
        ad = jnp.abs(d)
        sl1 = jnp.where(ad < 1.0, 0.5 * ad * ad, ad - 0.5)
        loc_sum = loc_sum + jnp.sum(jnp.where(pos, sl1, 0.0))

    conf0 = conf_ref[0, 0]
    mx = conf0
    for c in range(1, _C):
        mx = jnp.maximum(mx, conf_ref[0, c])
    s = jnp.exp(conf0 - mx)
    conf_lab = conf0
    for c in range(1, _C):
        cc = conf_ref[0, c]
        s = s + jnp.exp(cc - mx)
        conf_lab = jnp.where(label == c, cc, conf_lab)
    lse = jnp.log(s) + mx

    pos_ce_sum = jnp.sum(jnp.where(pos, lse - conf_lab, 0.0))
    neg_mask = jnp.logical_and(valid_anchor, jnp.logical_not(pos))
    loss_bg = jnp.where(neg_mask, lse - conf0, -jnp.inf)

    num_pos_i = num_pos.astype(jnp.int32)
    k = jnp.minimum(num_pos_i * _NEG_POS_RATIO, _A - num_pos_i)
    k_f = k.astype(jnp.float32)

    u = jax.lax.bitcast_convert_type(loss_bg, jnp.uint32)
    key = jnp.where(
        (u >> 31) == 1, ~u, u | jnp.uint32(0x80000000))

    def bit_body(i, p):
        cand = p | (jnp.uint32(1) << (jnp.uint32(31) - i.astype(jnp.uint32)))
        cnt = jnp.sum(jnp.where(key >= cand, 1.0, 0.0))
        return jnp.where(cnt >= k_f, cand, p)

    p = jax.lax.fori_loop(0, 32, bit_body, jnp.uint32(0))

    gt_mask = key > p
    cnt_gt = jnp.sum(jnp.where(gt_mask, 1.0, 0.0))
    sum_gt = jnp.sum(jnp.where(gt_mask, loss_bg, 0.0))
    thr = jnp.max(jnp.where(key == p, loss_bg, -jnp.inf))
    neg_sum = jnp.where(k > 0, sum_gt + thr * (k_f - cnt_gt), 0.0)

    cls_sum = pos_ce_sum + neg_sum

    @pl.when(b == 0)
    def _init():
        out_ref[0] = 0.0
        out_ref[1] = 0.0
        out_ref[2] = 0.0

    out_ref[0] += loc_sum
    out_ref[1] += cls_sum
    out_ref[2] += num_pos

    @pl.when(b == nb - 1)
    def _finalize():
        denom = 4.0 * jnp.maximum(1.0, out_ref[2])
        out_ref[0] = out_ref[0] / denom
        out_ref[1] = out_ref[1] / denom


@functools.partial(jax.jit, static_argnames=("interpret",))
def kernel(confidence, predicted_locations, gts, counts, anchors,
           interpret=False):
    B = confidence.shape[0]

    conf_t = jnp.moveaxis(confidence, 2, 1)
    conf_t = jnp.pad(conf_t, ((0, 0), (0, 0), (0, _A_PAD - _A)))
    conf_t = conf_t.reshape(B, _C, _R, _L)

    pred_t = jnp.moveaxis(predicted_locations, 2, 1)
    pred_t = jnp.pad(pred_t, ((0, 0), (0, 0), (0, _A_PAD - _A)))
    pred_t = pred_t.reshape(B, 4, _R, _L)

    ax1, ay1, ax2, ay2 = anchors[:, 0], anchors[:, 1], anchors[:, 2], anchors[:, 3]
    acx = (ax1 + ax2) * 0.5
    acy = (ay1 + ay2) * 0.5
    aw = ax2 - ax1
    ah = ay2 - ay1
    aarea = jnp.clip(aw, 0, None) * jnp.clip(ah, 0, None)
    anch = jnp.stack([ax1, ay1, ax2, ay2, acx, acy, aw, ah, aarea])
    pad_vals = jnp.array([0, 0, 0, 0, 0, 0, 1, 1, 0], jnp.float32)
    anch = jnp.concatenate(
        [anch, jnp.broadcast_to(pad_vals[:, None], (9, _A_PAD - _A))], axis=1)
    anch = anch.reshape(9, _R, _L)

    out = pl.pallas_call(
        _mbl_body,
        grid=(B,),
        in_specs=[
            pl.BlockSpec((1, _C, _R, _L), lambda b: (b, 0, 0, 0)),
            pl.BlockSpec((1, 4, _R, _L), lambda b: (b, 0, 0, 0)),
            pl.BlockSpec((1, _G, 5), lambda b: (b, 0, 0),
                         memory_space=pltpu.SMEM),
            pl.BlockSpec((1, 1, 1), lambda b: (b, 0, 0),
                         memory_space=pltpu.SMEM),
            pl.BlockSpec((9, _R, _L), lambda b: (0, 0, 0)),
        ],
        out_specs=pl.BlockSpec((3,), lambda b: (0,),
                               memory_space=pltpu.SMEM),
        out_shape=jax.ShapeDtypeStruct((3,), jnp.float32),
        interpret=interpret,
    )(conf_t, pred_t, gts, counts.reshape(B, 1, 1), anch)

    return (out[0], out[1])


# speedup vs baseline: 12.2950x; 2.0593x over previous
import functools

import jax
import jax.numpy as jnp
from jax.experimental import pallas as pl
from jax.experimental.pallas import tpu as pltpu

_POS_THRESH = 0.5
_NEG_POS_RATIO = 3
_VAR_C = 0.1
_VAR_S = 0.2

_A = 8732
_G = 50
_C = 21
_R = 8
_L = 1152
_A_PAD = _R * _L


def _match_body(conf_ref, pred_ref, gts_ref, counts_ref, anch_ref,
                loss_ref, part_ref):
    count = counts_ref[0, 0, 0]

    ax1 = anch_ref[0]
    ay1 = anch_ref[1]
    ax2 = anch_ref[2]
    ay2 = anch_ref[3]
    acx = anch_ref[4]
    acy = anch_ref[5]
    aw = anch_ref[6]
    ah = anch_ref[7]
    aarea = anch_ref[8]

    ridx = jax.lax.broadcasted_iota(jnp.int32, (_R, _L), 0)
    cidx = jax.lax.broadcasted_iota(jnp.int32, (_R, _L), 1)
    aidx = ridx * _L + cidx
    valid_anchor = aidx < _A

    def gt_body(g, carry):
        best_iou, best_idx = carry
        gx1 = gts_ref[0, g, 0]
        gy1 = gts_ref[0, g, 1]
        gx2 = gts_ref[0, g, 2]
        gy2 = gts_ref[0, g, 3]

        lt_x = jnp.maximum(gx1, ax1)
        lt_y = jnp.maximum(gy1, ay1)
        rb_x = jnp.minimum(gx2, ax2)
        rb_y = jnp.minimum(gy2, ay2)
        inter = jnp.maximum(rb_x - lt_x, 0.0) * jnp.maximum(rb_y - lt_y, 0.0)
        garea = jnp.maximum(gx2 - gx1, 0.0) * jnp.maximum(gy2 - gy1, 0.0)
        union = garea + aarea - inter
        iou = inter / jnp.maximum(union, 1e-8)

        upd = iou > best_iou
        best_idx = jnp.where(upd, g, best_idx)
        best_iou = jnp.where(upd, iou, best_iou)

        m = jnp.max(iou)
        astar = jnp.min(jnp.where(iou == m, aidx, jnp.int32(2**30)))
        fm = aidx == astar
        best_idx = jnp.where(fm, g, best_idx)
        best_iou = jnp.where(fm, jnp.float32(2.0), best_iou)
        return best_iou, best_idx

    best_iou0 = jnp.full((_R, _L), -1.0, dtype=jnp.float32)
    best_idx0 = jnp.zeros((_R, _L), dtype=jnp.int32)
    best_iou, best_idx = jax.lax.fori_loop(
        0, count, gt_body, (best_iou0, best_idx0))

    def gather_body(g, carry):
        lab, bx1, by1, bx2, by2 = carry
        m = best_idx == g
        lab = jnp.where(m, gts_ref[0, g, 4], lab)
        bx1 = jnp.where(m, gts_ref[0, g, 0], bx1)
        by1 = jnp.where(m, gts_ref[0, g, 1], by1)
        bx2 = jnp.where(m, gts_ref[0, g, 2], bx2)
        by2 = jnp.where(m, gts_ref[0, g, 3], by2)
        return lab, bx1, by1, bx2, by2

    z = jnp.zeros((_R, _L), dtype=jnp.float32)
    lab, bx1, by1, bx2, by2 = jax.lax.fori_loop(
        0, count, gather_body, (z, z, z, z, z))

    label = jnp.where(best_iou < _POS_THRESH, 0.0, lab)
    pos = label > 0.0
    num_pos = jnp.sum(jnp.where(pos, 1.0, 0.0))

    gcx = (bx1 + bx2) * 0.5
    gcy = (by1 + by2) * 0.5
    gw = bx2 - bx1
    gh = by2 - by1
    t0 = (gcx - acx) / (aw * _VAR_C)
    t1 = (gcy - acy) / (ah * _VAR_C)
    t2 = jnp.log(jnp.maximum(gw, 1e-8) / aw) / _VAR_S
    t3 = jnp.log(jnp.maximum(gh, 1e-8) / ah) / _VAR_S

    loc_sum = jnp.float32(0.0)
    for j, t in enumerate((t0, t1, t2, t3)):
        d = pred_ref[0, j] - t
        ad = jnp.abs(d)
        sl1 = jnp.where(ad < 1.0, 0.5 * ad * ad, ad - 0.5)
        loc_sum = loc_sum + jnp.sum(jnp.where(pos, sl1, 0.0))

    conf0 = conf_ref[0, 0]
    mx = conf0
    for c in range(1, _C):
        mx = jnp.maximum(mx, conf_ref[0, c])
    s = jnp.exp(conf0 - mx)
    conf_lab = conf0
    for c in range(1, _C):
        cc = conf_ref[0, c]
        s = s + jnp.exp(cc - mx)
        conf_lab = jnp.where(label == c, cc, conf_lab)
    lse = jnp.log(s) + mx

    pos_ce_sum = jnp.sum(jnp.where(pos, lse - conf_lab, 0.0))
    neg_mask = jnp.logical_and(valid_anchor, jnp.logical_not(pos))
    loss_ref[0] = jnp.where(neg_mask, lse - conf0, -jnp.inf)

    part_ref[0, 0, 0] = loc_sum
    part_ref[0, 0, 1] = pos_ce_sum
    part_ref[0, 0, 2] = num_pos


def _mine_body(loss_ref, part_ref, out_ref):
    lb = loss_ref[...]
    parts = part_ref[...]
    num_pos = parts[:, 2:3]

    k = jnp.minimum(num_pos * _NEG_POS_RATIO, _A - num_pos)

    u = jax.lax.bitcast_convert_type(lb, jnp.uint32)
    key = jnp.where((u >> 31) == 1, ~u, u | jnp.uint32(0x80000000))

    def bit_body(i, p):
        cand = p | (jnp.uint32(1) << (jnp.uint32(31) - i.astype(jnp.uint32)))
        cnt = jnp.sum(jnp.where(key >= cand, 1.0, 0.0), axis=1, keepdims=True)
        return jnp.where(cnt >= k, cand, p)

    p0 = jnp.zeros(num_pos.shape, dtype=jnp.uint32)
    p = jax.lax.fori_loop(0, 32, bit_body, p0)

    gt_mask = key > p
    cnt_gt = jnp.sum(jnp.where(gt_mask, 1.0, 0.0), axis=1, keepdims=True)
    sum_gt = jnp.sum(jnp.where(gt_mask, lb, 0.0), axis=1, keepdims=True)
    thr = jnp.max(jnp.where(key == p, lb, -jnp.inf), axis=1, keepdims=True)
    neg_sum = jnp.where(k > 0, sum_gt + thr * (k - cnt_gt), 0.0)

    loc_total = jnp.sum(parts[:, 0:1])
    cls_total = jnp.sum(parts[:, 1:2]) + jnp.sum(neg_sum)
    np_total = jnp.sum(num_pos)
    denom = 4.0 * jnp.maximum(1.0, np_total)
    out_ref[0] = loc_total / denom
    out_ref[1] = cls_total / denom


@functools.partial(jax.jit, static_argnames=("interpret",))
def kernel(confidence, predicted_locations, gts, counts, anchors,
           interpret=False):
    B = confidence.shape[0]

    conf_t = jnp.moveaxis(confidence, 2, 1)
    conf_t = jnp.pad(conf_t, ((0, 0), (0, 0), (0, _A_PAD - _A)))
    conf_t = conf_t.reshape(B, _C, _R, _L)

    pred_t = jnp.moveaxis(predicted_locations, 2, 1)
    pred_t = jnp.pad(pred_t, ((0, 0), (0, 0), (0, _A_PAD - _A)))
    pred_t = pred_t.reshape(B, 4, _R, _L)

    ax1, ay1, ax2, ay2 = anchors[:, 0], anchors[:, 1], anchors[:, 2], anchors[:, 3]
    acx = (ax1 + ax2) * 0.5
    acy = (ay1 + ay2) * 0.5
    aw = ax2 - ax1
    ah = ay2 - ay1
    aarea = jnp.clip(aw, 0, None) * jnp.clip(ah, 0, None)
    anch = jnp.stack([ax1, ay1, ax2, ay2, acx, acy, aw, ah, aarea])
    pad_vals = jnp.array([0, 0, 0, 0, 0, 0, 1, 1, 0], jnp.float32)
    anch = jnp.concatenate(
        [anch, jnp.broadcast_to(pad_vals[:, None], (9, _A_PAD - _A))], axis=1)
    anch = anch.reshape(9, _R, _L)

    loss_rows, partials = pl.pallas_call(
        _match_body,
        grid=(B,),
        in_specs=[
            pl.BlockSpec((1, _C, _R, _L), lambda b: (b, 0, 0, 0)),
            pl.BlockSpec((1, 4, _R, _L), lambda b: (b, 0, 0, 0)),
            pl.BlockSpec((1, _G, 5), lambda b: (b, 0, 0),
                         memory_space=pltpu.SMEM),
            pl.BlockSpec((1, 1, 1), lambda b: (b, 0, 0),
                         memory_space=pltpu.SMEM),
            pl.BlockSpec((9, _R, _L), lambda b: (0, 0, 0)),
        ],
        out_specs=[
            pl.BlockSpec((1, _R, _L), lambda b: (b, 0, 0)),
            pl.BlockSpec((1, 1, 3), lambda b: (b, 0, 0),
                         memory_space=pltpu.SMEM),
        ],
        out_shape=[
            jax.ShapeDtypeStruct((B, _R, _L), jnp.float32),
            jax.ShapeDtypeStruct((B, 1, 3), jnp.float32),
        ],
        interpret=interpret,
    )(conf_t, pred_t, gts, counts.reshape(B, 1, 1), anch)

    out = pl.pallas_call(
        _mine_body,
        in_specs=[
            pl.BlockSpec((B, _A_PAD), lambda: (0, 0)),
            pl.BlockSpec((B, 3), lambda: (0, 0)),
        ],
        out_specs=pl.BlockSpec((3,), lambda: (0,), memory_space=pltpu.SMEM),
        out_shape=jax.ShapeDtypeStruct((3,), jnp.float32),
        interpret=interpret,
    )(loss_rows.reshape(B, _A_PAD), partials.reshape(B, 3))

    return (out[0], out[1])


# speedup vs baseline: 15.0956x; 1.2278x over previous
import functools

import jax
import jax.numpy as jnp
from jax.experimental import pallas as pl
from jax.experimental.pallas import tpu as pltpu

_POS_THRESH = 0.5
_NEG_POS_RATIO = 3
_VAR_C = 0.1
_VAR_S = 0.2

_A = 8732
_G = 50
_C = 21
_R = 8
_L = 1152
_A_PAD = _R * _L


def _match_body(conf_ref, pred_ref, gts_ref, counts_ref, anch_ref,
                loss_ref, part_ref):
    count = counts_ref[0, 0, 0]

    ax1 = anch_ref[0]
    ay1 = anch_ref[1]
    ax2 = anch_ref[2]
    ay2 = anch_ref[3]
    acx = anch_ref[4]
    acy = anch_ref[5]
    aw = anch_ref[6]
    ah = anch_ref[7]
    aarea = anch_ref[8]

    ridx = jax.lax.broadcasted_iota(jnp.int32, (_R, _L), 0)
    cidx = jax.lax.broadcasted_iota(jnp.int32, (_R, _L), 1)
    aidx = ridx * _L + cidx
    valid_anchor = aidx < _A

    UNROLL = 4

    def one_gt(g):
        gx1 = gts_ref[0, g, 0]
        gy1 = gts_ref[0, g, 1]
        gx2 = gts_ref[0, g, 2]
        gy2 = gts_ref[0, g, 3]

        lt_x = jnp.maximum(gx1, ax1)
        lt_y = jnp.maximum(gy1, ay1)
        rb_x = jnp.minimum(gx2, ax2)
        rb_y = jnp.minimum(gy2, ay2)
        inter = jnp.maximum(rb_x - lt_x, 0.0) * jnp.maximum(rb_y - lt_y, 0.0)
        garea = jnp.maximum(gx2 - gx1, 0.0) * jnp.maximum(gy2 - gy1, 0.0)
        union = garea + aarea - inter
        iou = inter / jnp.maximum(union, 1e-8)

        m = jnp.max(iou, axis=(0, 1), keepdims=True)
        astar = jnp.min(jnp.where(iou == m, aidx, jnp.int32(2**30)),
                        axis=(0, 1), keepdims=True)
        return iou, astar

    def gt_block(i, carry):
        best_iou, lab, bx1, by1, bx2, by2 = carry
        g0 = i * UNROLL
        rows = [one_gt(jnp.minimum(g0 + j, count - 1)) for j in range(UNROLL)]
        for j, (iou, astar) in enumerate(rows):
            g = jnp.minimum(g0 + j, count - 1)
            valid = (g0 + j) < count
            upd = jnp.logical_and(valid, iou > best_iou)
            fm = jnp.logical_and(valid, aidx == astar)
            sel = jnp.logical_or(upd, fm)
            lab = jnp.where(sel, gts_ref[0, g, 4], lab)
            bx1 = jnp.where(sel, gts_ref[0, g, 0], bx1)
            by1 = jnp.where(sel, gts_ref[0, g, 1], by1)
            bx2 = jnp.where(sel, gts_ref[0, g, 2], bx2)
            by2 = jnp.where(sel, gts_ref[0, g, 3], by2)
            best_iou = jnp.where(upd, iou, best_iou)
            best_iou = jnp.where(fm, jnp.float32(2.0), best_iou)
        return best_iou, lab, bx1, by1, bx2, by2

    z = jnp.zeros((_R, _L), dtype=jnp.float32)
    best_iou0 = jnp.full((_R, _L), -1.0, dtype=jnp.float32)
    n_blocks = (count + (UNROLL - 1)) // UNROLL
    best_iou, lab, bx1, by1, bx2, by2 = jax.lax.fori_loop(
        0, n_blocks, gt_block, (best_iou0, z, z, z, z, z))

    label = jnp.where(best_iou < _POS_THRESH, 0.0, lab)
    pos = label > 0.0
    num_pos = jnp.sum(jnp.where(pos, 1.0, 0.0))

    gcx = (bx1 + bx2) * 0.5
    gcy = (by1 + by2) * 0.5
    gw = bx2 - bx1
    gh = by2 - by1
    t0 = (gcx - acx) / (aw * _VAR_C)
    t1 = (gcy - acy) / (ah * _VAR_C)
    t2 = jnp.log(jnp.maximum(gw, 1e-8) / aw) / _VAR_S
    t3 = jnp.log(jnp.maximum(gh, 1e-8) / ah) / _VAR_S

    loc_sum = jnp.float32(0.0)
    for j, t in enumerate((t0, t1, t2, t3)):
        d = pred_ref[0, j] - t
        ad = jnp.abs(d)
        sl1 = jnp.where(ad < 1.0, 0.5 * ad * ad, ad - 0.5)
        loc_sum = loc_sum + jnp.sum(jnp.where(pos, sl1, 0.0))

    conf0 = conf_ref[0, 0]
    mx = conf0
    for c in range(1, _C):
        mx = jnp.maximum(mx, conf_ref[0, c])
    s = jnp.exp(conf0 - mx)
    conf_lab = conf0
    for c in range(1, _C):
        cc = conf_ref[0, c]
        s = s + jnp.exp(cc - mx)
        conf_lab = jnp.where(label == c, cc, conf_lab)
    lse = jnp.log(s) + mx

    pos_ce_sum = jnp.sum(jnp.where(pos, lse - conf_lab, 0.0))
    neg_mask = jnp.logical_and(valid_anchor, jnp.logical_not(pos))
    loss_ref[0] = jnp.where(neg_mask, lse - conf0, -jnp.inf)

    part_ref[0, 0, 0] = loc_sum
    part_ref[0, 0, 1] = pos_ce_sum
    part_ref[0, 0, 2] = num_pos


def _mine_body(loss_ref, part_ref, out_ref):
    lb = loss_ref[...]
    parts = part_ref[...]
    num_pos = parts[:, 2:3]

    k = jnp.minimum(num_pos * _NEG_POS_RATIO, _A - num_pos)

    u = jax.lax.bitcast_convert_type(lb, jnp.uint32)
    key = jnp.where((u >> 31) == 1, ~u, u | jnp.uint32(0x80000000))

    def bit_body(i, p):
        cand = p | (jnp.uint32(1) << (jnp.uint32(31) - i.astype(jnp.uint32)))
        cnt = jnp.sum(jnp.where(key >= cand, 1.0, 0.0), axis=1, keepdims=True)
        return jnp.where(cnt >= k, cand, p)

    p0 = jnp.zeros(num_pos.shape, dtype=jnp.uint32)
    p = jax.lax.fori_loop(0, 32, bit_body, p0)

    gt_mask = key > p
    cnt_gt = jnp.sum(jnp.where(gt_mask, 1.0, 0.0), axis=1, keepdims=True)
    sum_gt = jnp.sum(jnp.where(gt_mask, lb, 0.0), axis=1, keepdims=True)
    thr = jnp.max(jnp.where(key == p, lb, -jnp.inf), axis=1, keepdims=True)
    neg_sum = jnp.where(k > 0, sum_gt + thr * (k - cnt_gt), 0.0)

    loc_total = jnp.sum(parts[:, 0:1])
    cls_total = jnp.sum(parts[:, 1:2]) + jnp.sum(neg_sum)
    np_total = jnp.sum(num_pos)
    denom = 4.0 * jnp.maximum(1.0, np_total)
    out_ref[0] = loc_total / denom
    out_ref[1] = cls_total / denom


@functools.partial(jax.jit, static_argnames=("interpret",))
def kernel(confidence, predicted_locations, gts, counts, anchors,
           interpret=False):
    B = confidence.shape[0]

    conf_t = jnp.moveaxis(confidence, 2, 1)
    conf_t = jnp.pad(conf_t, ((0, 0), (0, 0), (0, _A_PAD - _A)))
    conf_t = conf_t.reshape(B, _C, _R, _L)

    pred_t = jnp.moveaxis(predicted_locations, 2, 1)
    pred_t = jnp.pad(pred_t, ((0, 0), (0, 0), (0, _A_PAD - _A)))
    pred_t = pred_t.reshape(B, 4, _R, _L)

    ax1, ay1, ax2, ay2 = anchors[:, 0], anchors[:, 1], anchors[:, 2], anchors[:, 3]
    acx = (ax1 + ax2) * 0.5
    acy = (ay1 + ay2) * 0.5
    aw = ax2 - ax1
    ah = ay2 - ay1
    aarea = jnp.clip(aw, 0, None) * jnp.clip(ah, 0, None)
    anch = jnp.stack([ax1, ay1, ax2, ay2, acx, acy, aw, ah, aarea])
    pad_vals = jnp.array([0, 0, 0, 0, 0, 0, 1, 1, 0], jnp.float32)
    anch = jnp.concatenate(
        [anch, jnp.broadcast_to(pad_vals[:, None], (9, _A_PAD - _A))], axis=1)
    anch = anch.reshape(9, _R, _L)

    loss_rows, partials = pl.pallas_call(
        _match_body,
        grid=(B,),
        in_specs=[
            pl.BlockSpec((1, _C, _R, _L), lambda b: (b, 0, 0, 0)),
            pl.BlockSpec((1, 4, _R, _L), lambda b: (b, 0, 0, 0)),
            pl.BlockSpec((1, _G, 5), lambda b: (b, 0, 0),
                         memory_space=pltpu.SMEM),
            pl.BlockSpec((1, 1, 1), lambda b: (b, 0, 0),
                         memory_space=pltpu.SMEM),
            pl.BlockSpec((9, _R, _L), lambda b: (0, 0, 0)),
        ],
        out_specs=[
            pl.BlockSpec((1, _R, _L), lambda b: (b, 0, 0)),
            pl.BlockSpec((1, 1, 3), lambda b: (b, 0, 0),
                         memory_space=pltpu.SMEM),
        ],
        out_shape=[
            jax.ShapeDtypeStruct((B, _R, _L), jnp.float32),
            jax.ShapeDtypeStruct((B, 1, 3), jnp.float32),
        ],
        interpret=interpret,
    )(conf_t, pred_t, gts, counts.reshape(B, 1, 1), anch)

    out = pl.pallas_call(
        _mine_body,
        in_specs=[
            pl.BlockSpec((B, _A_PAD), lambda: (0, 0)),
            pl.BlockSpec((B, 3), lambda: (0, 0)),
        ],
        out_specs=pl.BlockSpec((3,), lambda: (0,), memory_space=pltpu.SMEM),
        out_shape=jax.ShapeDtypeStruct((3,), jnp.float32),
        interpret=interpret,
    )(loss_rows.reshape(B, _A_PAD), partials.reshape(B, 3))

    return (out[0], out[1])
